# baseline (speedup 1.0000x reference)
import functools

import jax
import jax.numpy as jnp
from jax import lax
from jax.experimental import pallas as pl
from jax.experimental.pallas import tpu as pltpu
from jax.experimental.pallas import tpu_sc as plsc

_B = 4096 * 200
_D = 64
_NW = 32
_CH = 128
_K = 8
_CPW = _B // _NW // _CH
_GROUPS = _CPW // _K

_mesh = plsc.VectorSubcoreMesh(core_axis_name="c", subcore_axis_name="s")


@functools.partial(
    pl.kernel,
    mesh=_mesh,
    out_type=jax.ShapeDtypeStruct((_B, _D), jnp.float32),
    scratch_types=[
        pltpu.VMEM((_CPW, _CH), jnp.int32),
        pltpu.VMEM((_K * _CH, _D), jnp.float32),
        pltpu.SemaphoreType.DMA,
    ],
    compiler_params=pltpu.CompilerParams(use_tc_tiling_on_sc=False),
)
def _gather_kernel(x_hbm, w_hbm, out_hbm, idx_v, rows_v, sem):
    wid = lax.axis_index("s") * 2 + lax.axis_index("c")
    chunk0 = wid * _CPW
    pltpu.sync_copy(x_hbm.at[pl.ds(chunk0, _CPW)], idx_v)

    def body(g, carry):
        copies = []
        for j in range(_K):
            c = pltpu.async_copy(
                w_hbm.at[idx_v.at[g * _K + j]],
                rows_v.at[pl.ds(j * _CH, _CH)],
                sem,
            )
            copies.append(c)
        for c in copies:
            c.wait()
        row0 = (chunk0 + g * _K) * _CH
        pltpu.sync_copy(rows_v, out_hbm.at[pl.ds(row0, _K * _CH)])
        return carry

    lax.fori_loop(0, _GROUPS, body, 0)


def kernel(x, weight):
    x_flat = x.reshape(_B // _CH, _CH).astype(jnp.int32)
    out = _gather_kernel(x_flat, weight)
    return out.reshape(x.shape + (_D,))


# speedup vs baseline: 1.0062x; 1.0062x over previous
import functools

import jax
import jax.numpy as jnp
from jax import lax
from jax.experimental import pallas as pl
from jax.experimental.pallas import tpu as pltpu
from jax.experimental.pallas import tpu_sc as plsc

_B = 4096 * 200
_D = 64
_NW = 32
_CH = 128
_K = 5
_GR = _K * _CH
_CPW = _B // _NW // _CH
_NGRP = _CPW // _K
_NIT = _NGRP // 2

_mesh = plsc.VectorSubcoreMesh(core_axis_name="c", subcore_axis_name="s")


@functools.partial(
    pl.kernel,
    mesh=_mesh,
    out_type=jax.ShapeDtypeStruct((_B, _D), jnp.float32),
    scratch_types=[
        pltpu.VMEM((_CPW, _CH), jnp.int32),
        pltpu.VMEM((_GR, _D), jnp.float32),
        pltpu.VMEM((_GR, _D), jnp.float32),
        pltpu.SemaphoreType.DMA,
        pltpu.SemaphoreType.DMA,
        pltpu.SemaphoreType.DMA,
    ],
    compiler_params=pltpu.CompilerParams(use_tc_tiling_on_sc=False),
)
def _gather_kernel(x_hbm, w_hbm, out_hbm, idx_v, rows0_v, rows1_v,
                   gsem, wsem0, wsem1):
    wid = lax.axis_index("s") * 2 + lax.axis_index("c")
    chunk0 = wid * _CPW
    pltpu.sync_copy(x_hbm.at[pl.ds(chunk0, _CPW)], idx_v)

    def fire(g, rows_v):
        return [
            pltpu.async_copy(
                w_hbm.at[idx_v.at[g * _K + j]],
                rows_v.at[pl.ds(j * _CH, _CH)],
                gsem,
            )
            for j in range(_K)
        ]

    def body(i, carry):
        ga = 2 * i
        gb = 2 * i + 1
        @pl.when(i > 0)
        def _():
            pltpu.make_async_copy(
                rows0_v, out_hbm.at[pl.ds(0, _GR)], wsem0).wait()
        ca = fire(ga, rows0_v)
        @pl.when(i > 0)
        def _():
            pltpu.make_async_copy(
                rows1_v, out_hbm.at[pl.ds(0, _GR)], wsem1).wait()
        for c in ca:
            c.wait()
        pltpu.async_copy(
            rows0_v, out_hbm.at[pl.ds((chunk0 + ga * _K) * _CH, _GR)], wsem0)
        cb = fire(gb, rows1_v)
        for c in cb:
            c.wait()
        pltpu.async_copy(
            rows1_v, out_hbm.at[pl.ds((chunk0 + gb * _K) * _CH, _GR)], wsem1)
        return carry

    lax.fori_loop(0, _NIT, body, 0)
    pltpu.make_async_copy(rows0_v, out_hbm.at[pl.ds(0, _GR)], wsem0).wait()
    pltpu.make_async_copy(rows1_v, out_hbm.at[pl.ds(0, _GR)], wsem1).wait()


def kernel(x, weight):
    x_flat = x.reshape(_B // _CH, _CH).astype(jnp.int32)
    out = _gather_kernel(x_flat, weight)
    return out.reshape(x.shape + (_D,))
